# Initial kernel scaffold; baseline (speedup 1.0000x reference)
import jax
import jax.numpy as jnp
from jax.experimental import pallas as pl


def kernel(x, init_centroids):
    raise NotImplementedError("write your pallas kernel here")


# speedup vs baseline: 7.1513x; 7.1513x over previous
import jax
import jax.numpy as jnp
from jax.experimental import pallas as pl

_K = 512
_D = 32
_N = 4096
_ITERS = 1000


def _kmeans_body(xa_ref, c0_ref, cents_ref, idxs_ref, nums_ref):
    xa = xa_ref[...]
    iota = jax.lax.broadcasted_iota(jnp.int32, (_N, _K), 1)

    def body(_, carry):
        cents, _, _ = carry
        cnorm = jnp.sum(cents * cents, axis=1, keepdims=True)
        caug = jnp.concatenate([cents * (-2.0), cnorm], axis=1)
        score = jax.lax.dot_general(
            xa, caug, (((1,), (1,)), ((), ())),
            preferred_element_type=jnp.float32,
            precision=jax.lax.Precision.HIGHEST)
        mind = jnp.min(score, axis=1, keepdims=True)
        idx = jnp.min(jnp.where(score == mind, iota, _K),
                      axis=1, keepdims=True).astype(jnp.int32)
        onehot = (iota == idx).astype(jnp.float32)
        sums = jax.lax.dot_general(
            onehot, xa, (((0,), (0,)), ((), ())),
            preferred_element_type=jnp.float32,
            precision=jax.lax.Precision.HIGHEST)
        counts = sums[:, _D:]
        new_cents = sums[:, :_D] / (counts + 1e-6)
        return new_cents, idx, counts

    init = (c0_ref[...],
            jnp.zeros((_N, 1), jnp.int32),
            jnp.zeros((_K, 1), jnp.float32))
    cents, idx, counts = jax.lax.fori_loop(0, _ITERS, body, init)
    cents_ref[...] = cents
    idxs_ref[...] = idx
    nums_ref[...] = counts


def kernel(x, init_centroids):
    xa = jnp.concatenate([x, jnp.ones((x.shape[0], 1), x.dtype)], axis=1)
    cents, idx, nums = pl.pallas_call(
        _kmeans_body,
        out_shape=[
            jax.ShapeDtypeStruct((_K, _D), jnp.float32),
            jax.ShapeDtypeStruct((_N, 1), jnp.int32),
            jax.ShapeDtypeStruct((_K, 1), jnp.float32),
        ],
    )(xa, init_centroids)
    return cents, idx.reshape(_N), nums


# speedup vs baseline: 22.6600x; 3.1686x over previous
import jax
import jax.numpy as jnp
from jax.experimental import pallas as pl

_K = 512
_D = 32
_N = 4096
_ITERS = 1000


def _kmeans_body(xat_ref, xb_ref, c0_ref, cents_ref, idxs_ref, nums_ref):
    xat = xat_ref[...]
    xb = xb_ref[...]
    iota0 = jax.lax.broadcasted_iota(jnp.int32, (_K, _N), 0)

    def body(_, carry):
        cents, _, _ = carry
        cnorm = jnp.sum(cents * cents, axis=1, keepdims=True)
        caug = jnp.concatenate([cents * (-2.0), cnorm], axis=1)
        scoret = jax.lax.dot_general(
            caug, xat, (((1,), (0,)), ((), ())),
            preferred_element_type=jnp.float32,
            precision=jax.lax.Precision.HIGHEST)
        mind = jnp.min(scoret, axis=0, keepdims=True)
        idx = jnp.min(jnp.where(scoret == mind, iota0, _K),
                      axis=0, keepdims=True).astype(jnp.int32)
        onehott = (iota0 == idx).astype(jnp.bfloat16)
        s = jax.lax.dot_general(
            onehott, xb, (((1,), (0,)), ((), ())),
            preferred_element_type=jnp.float32)
        counts = s[:, _D:_D + 1]
        new_cents = (s[:, :_D] + s[:, 64:64 + _D]) / (counts + 1e-6)
        return new_cents, idx, counts

    init = (c0_ref[...],
            jnp.zeros((1, _N), jnp.int32),
            jnp.zeros((_K, 1), jnp.float32))
    cents, idx, counts = jax.lax.fori_loop(0, _ITERS, body, init)
    cents_ref[...] = cents
    idxs_ref[...] = idx
    nums_ref[...] = counts


def kernel(x, init_centroids):
    n = x.shape[0]
    ones = jnp.ones((n, 1), x.dtype)
    xat = jnp.concatenate([x, ones], axis=1).T
    xhi = x.astype(jnp.bfloat16)
    xlo = (x - xhi.astype(jnp.float32)).astype(jnp.bfloat16)
    zpad = jnp.zeros((n, 31), jnp.bfloat16)
    xb = jnp.concatenate(
        [xhi, ones.astype(jnp.bfloat16), zpad, xlo, zpad, jnp.zeros((n, 1), jnp.bfloat16)],
        axis=1)
    cents, idx, nums = pl.pallas_call(
        _kmeans_body,
        out_shape=[
            jax.ShapeDtypeStruct((_K, _D), jnp.float32),
            jax.ShapeDtypeStruct((1, _N), jnp.int32),
            jax.ShapeDtypeStruct((_K, 1), jnp.float32),
        ],
    )(xat, xb, init_centroids)
    return cents, idx.reshape(_N), nums


# speedup vs baseline: 49.4862x; 2.1839x over previous
import jax
import jax.numpy as jnp
from jax.experimental import pallas as pl

_K = 512
_D = 32
_N = 4096
_ITERS = 1000


def _kmeans_body(xdist_ref, xb_ref, c0_ref, cents_ref, idxs_ref, nums_ref):
    xdist = xdist_ref[...]
    xb = xb_ref[...]
    iotaf = jax.lax.broadcasted_iota(jnp.int32, (_K, _N), 0).astype(jnp.float32)

    def body(_, carry):
        cents, _, _ = carry
        cn = jnp.sum(cents * cents, axis=1, keepdims=True)
        chi = cents.astype(jnp.bfloat16).astype(jnp.float32)
        cmid = (cents - chi).astype(jnp.bfloat16).astype(jnp.float32)
        clo = (cents - chi - cmid).astype(jnp.bfloat16).astype(jnp.float32)
        cn1 = cn.astype(jnp.bfloat16).astype(jnp.float32)
        cn2 = (cn - cn1).astype(jnp.bfloat16).astype(jnp.float32)
        cn3 = (cn - cn1 - cn2).astype(jnp.bfloat16).astype(jnp.float32)
        cn4 = (cn - cn1 - cn2 - cn3).astype(jnp.bfloat16).astype(jnp.float32)
        caug = jnp.concatenate(
            [chi * -2.0, chi * -2.0, cmid * -2.0, chi * -2.0, clo * -2.0,
             cmid * -2.0, cn1, cn2, cn3, cn4,
             jnp.zeros((_K, 256 - 6 * _D - 4), jnp.float32)],
            axis=1).astype(jnp.bfloat16)
        scoret = jax.lax.dot_general(
            caug, xdist, (((1,), (0,)), ((), ())),
            preferred_element_type=jnp.float32)
        mind = jnp.min(scoret, axis=0, keepdims=True)
        idxf = jnp.min(jnp.where(scoret == mind, iotaf, float(_K)),
                       axis=0, keepdims=True)
        onehott = (iotaf == idxf).astype(jnp.bfloat16)
        s = jax.lax.dot_general(
            onehott, xb, (((1,), (0,)), ((), ())),
            preferred_element_type=jnp.float32)
        counts = s[:, _D:_D + 1]
        new_cents = (s[:, :_D] + s[:, 64:64 + _D]) / (counts + 1e-6)
        return new_cents, idxf, counts

    init = (c0_ref[...],
            jnp.zeros((1, _N), jnp.float32),
            jnp.zeros((_K, 1), jnp.float32))
    cents, idxf, counts = jax.lax.fori_loop(0, _ITERS, body, init)
    cents_ref[...] = cents
    idxs_ref[...] = idxf.astype(jnp.int32)
    nums_ref[...] = counts


def kernel(x, init_centroids):
    n = x.shape[0]
    f32 = jnp.float32
    bf = jnp.bfloat16
    xhi = x.astype(bf).astype(f32)
    xmid = (x - xhi).astype(bf).astype(f32)
    xlo = (x - xhi - xmid).astype(bf).astype(f32)
    ones = jnp.ones((n, 1), f32)
    xdist = jnp.concatenate(
        [xhi, xmid, xhi, xlo, xhi, xmid, ones, ones, ones, ones,
         jnp.zeros((n, 256 - 6 * _D - 4), f32)], axis=1).T.astype(bf)
    xb = jnp.concatenate(
        [xhi, ones, jnp.zeros((n, 31), f32), xlo + xmid,
         jnp.zeros((n, 32), f32)], axis=1).astype(bf)
    cents, idx, nums = pl.pallas_call(
        _kmeans_body,
        out_shape=[
            jax.ShapeDtypeStruct((_K, _D), jnp.float32),
            jax.ShapeDtypeStruct((1, _N), jnp.int32),
            jax.ShapeDtypeStruct((_K, 1), jnp.float32),
        ],
    )(xdist, xb, init_centroids)
    return cents, idx.reshape(_N), nums
